# baseline (speedup 1.0000x reference)
import functools

import jax
import jax.numpy as jnp
from jax import lax
from jax.experimental import pallas as pl
from jax.experimental.pallas import tpu as pltpu
from jax.experimental.pallas import tpu_sc as plsc

_EMB = 64
_NDEP = 46
_NEG = 5
_NW = 32
_CHUNK = 512
_BLK = 512


def _make_sc_gather(B):
    nu = B // _NW // _CHUNK
    nv = (_NEG + 1) * B // _NW // _CHUNK
    mesh = plsc.VectorSubcoreMesh(core_axis_name="c", subcore_axis_name="s")

    def body(u_hbm, v_hbm, uidx_hbm, vidx_hbm, uout_hbm, vout_hbm,
             idx_v, rows_v, sem):
        wid = lax.axis_index("s") * 2 + lax.axis_index("c")
        ubase = wid * (B // _NW)
        for c in range(nu):
            off = ubase + c * _CHUNK
            pltpu.sync_copy(uidx_hbm.at[pl.ds(off, _CHUNK)], idx_v)
            pltpu.async_copy(u_hbm.at[idx_v], rows_v, sem).wait()
            pltpu.sync_copy(rows_v, uout_hbm.at[pl.ds(off, _CHUNK)])
        vbase = wid * ((_NEG + 1) * B // _NW)
        for c in range(nv):
            off = vbase + c * _CHUNK
            pltpu.sync_copy(vidx_hbm.at[pl.ds(off, _CHUNK)], idx_v)
            pltpu.async_copy(v_hbm.at[idx_v], rows_v, sem).wait()
            pltpu.sync_copy(rows_v, vout_hbm.at[pl.ds(off, _CHUNK)])

    return pl.kernel(
        body,
        mesh=mesh,
        compiler_params=pltpu.CompilerParams(use_tc_tiling_on_sc=False),
        out_type=[jax.ShapeDtypeStruct((B, _EMB), jnp.float32),
                  jax.ShapeDtypeStruct(((_NEG + 1) * B, _EMB), jnp.float32)],
        scratch_types=[pltpu.VMEM((_CHUNK,), jnp.int32),
                       pltpu.VMEM((_CHUNK, _EMB), jnp.float32),
                       pltpu.SemaphoreType.DMA],
    )


def _logsig(x):
    return jnp.minimum(x, 0.0) - jnp.log(1.0 + jnp.exp(-jnp.abs(x)))


def _tc_body(dep_ref, uw_ref, ow_ref, nz_ref, w_ref, out_ref):
    blk = uw_ref.shape[0]
    p = jnp.dot(uw_ref[...], w_ref[...], preferred_element_type=jnp.float32)
    kid = lax.broadcasted_iota(jnp.int32, (blk, _NDEP * _EMB), 1) >> 6
    masked = jnp.where(kid == dep_ref[...], p, 0.0)
    tin = masked[:, 0:_EMB]
    for k in range(1, _NDEP):
        tin = tin + masked[:, k * _EMB:(k + 1) * _EMB]
    vec_dot = jnp.sum(tin * ow_ref[...], axis=1, keepdims=True)
    total = jnp.sum(_logsig(vec_dot))
    for n in range(_NEG):
        dn = jnp.sum(tin * nz_ref[:, n * _EMB:(n + 1) * _EMB], axis=1,
                     keepdims=True)
        total = total + jnp.sum(_logsig(-dn))

    @pl.when(pl.program_id(0) == 0)
    def _init():
        out_ref[0, 0] = 0.0

    out_ref[0, 0] += total


def _tc_loss(dep2, uw, ow, nz, wcols):
    B = uw.shape[0]
    grid = B // _BLK
    return pl.pallas_call(
        _tc_body,
        grid=(grid,),
        in_specs=[
            pl.BlockSpec((_BLK, 1), lambda i: (i, 0)),
            pl.BlockSpec((_BLK, _EMB), lambda i: (i, 0)),
            pl.BlockSpec((_BLK, _EMB), lambda i: (i, 0)),
            pl.BlockSpec((_BLK, _NEG * _EMB), lambda i: (i, 0)),
            pl.BlockSpec((_EMB, _NDEP * _EMB), lambda i: (0, 0)),
        ],
        out_specs=pl.BlockSpec(memory_space=pltpu.MemorySpace.SMEM),
        out_shape=jax.ShapeDtypeStruct((1, 1), jnp.float32),
    )(dep2, uw, ow, nz, wcols)


def kernel(input_label, out_label, dep_label, use_given, u_emb, v_emb,
           dep_mxs):
    B = out_label.shape[0]
    vidx = jnp.concatenate([out_label, use_given.reshape(-1)])
    uw, vw = _make_sc_gather(B)(u_emb, v_emb, input_label, vidx)
    ow = vw[:B]
    nz = vw[B:].reshape(B, _NEG * _EMB)
    wcols = jnp.transpose(dep_mxs.reshape(_NDEP, _EMB, _EMB),
                          (1, 0, 2)).reshape(_EMB, _NDEP * _EMB)
    res = _tc_loss(dep_label.reshape(B, 1), uw, ow, nz, wcols)
    return -res[0, 0] / B
